# baseline (speedup 1.0000x reference)
import functools

import jax
import jax.numpy as jnp
from jax import lax
from jax.experimental import pallas as pl
from jax.experimental.pallas import tpu as pltpu
from jax.experimental.pallas import tpu_sc as plsc

N = 10000
D = 128
NP = 10240
RB = 1280
ROWS_PER_TILE = NP // 16

K = 128
NW = 32
CH = 84
SLAB = 28
EP = NW * CH * K


def _t1_body(x_ref, w_ref, as_ref, ad_ref, h_ref, asrc_ref, adst_ref, ms_ref):
    i = pl.program_id(0)
    h = jnp.dot(x_ref[...], w_ref[...], preferred_element_type=jnp.float32)
    h_ref[...] = h
    asv = jnp.sum(h * as_ref[...], axis=1, keepdims=True)
    adv = jnp.sum(h * ad_ref[...], axis=1, keepdims=True)
    asrc_ref[...] = asv
    adst_ref[...] = adv

    @pl.when(i == 0)
    def _():
        ms_ref[...] = jnp.full((1, 1), -jnp.inf, jnp.float32)

    ms_ref[...] = jnp.maximum(ms_ref[...], jnp.max(asv))


def _t1(x, w, a_s, a_d):
    return pl.pallas_call(
        _t1_body,
        grid=(NP // RB,),
        in_specs=[
            pl.BlockSpec((RB, D), lambda i: (i, 0)),
            pl.BlockSpec((D, D), lambda i: (0, 0)),
            pl.BlockSpec((1, D), lambda i: (0, 0)),
            pl.BlockSpec((1, D), lambda i: (0, 0)),
        ],
        out_specs=[
            pl.BlockSpec((RB, D), lambda i: (i, 0)),
            pl.BlockSpec((RB, 1), lambda i: (i, 0)),
            pl.BlockSpec((RB, 1), lambda i: (i, 0)),
            pl.BlockSpec((1, 1), lambda i: (0, 0)),
        ],
        out_shape=[
            jax.ShapeDtypeStruct((NP, D), jnp.float32),
            jax.ShapeDtypeStruct((NP, 1), jnp.float32),
            jax.ShapeDtypeStruct((NP, 1), jnp.float32),
            jax.ShapeDtypeStruct((1, 1), jnp.float32),
        ],
    )(x, w, a_s, a_d)


def _lrelu(v):
    return jnp.where(v > 0, v, 0.2 * v)


def _sck1_body(asrc_hbm, adst_hbm, ms_hbm, src_hbm, dst_hbm, ex_out,
               den_out, asrc_v, adst_v, ms_v, srcb, dstb, exb, den_v):
    cid = lax.axis_index("c")
    sid = lax.axis_index("s")
    wid = cid * 16 + sid

    pltpu.sync_copy(asrc_hbm, asrc_v)
    pltpu.sync_copy(adst_hbm, adst_v)
    pltpu.sync_copy(ms_hbm, ms_v)
    pltpu.sync_copy(src_hbm.at[wid], srcb)
    pltpu.sync_copy(dst_hbm.at[wid], dstb)

    def zden(i, _):
        den_v[pl.ds(i * 16, 16)] = jnp.zeros((16,), jnp.float32)
        return _

    lax.fori_loop(0, NP // 16, zden, None)

    ms_vec = ms_v[...]

    def grp_body(g, _):
        s16 = srcb[pl.ds(g * 16, 16)]
        d16 = dstb[pl.ds(g * 16, 16)]
        a_s = plsc.load_gather(asrc_v, [s16])
        a_d = plsc.load_gather(adst_v, [d16])
        e = _lrelu(a_s + a_d)
        shift = _lrelu(ms_vec + a_d)
        ex = jnp.exp(e - shift)
        exb[pl.ds(g * 16, 16)] = ex
        plsc.addupdate_scatter(den_v, [d16], ex)
        return _

    lax.fori_loop(0, (CH * K) // 16, grp_body, None)
    pltpu.sync_copy(exb, ex_out.at[wid])
    pltpu.sync_copy(den_v, den_out.at[wid])


def _sc_scalar_pass(asrc, adst, ms16, srcf, dstf):
    mesh = plsc.VectorSubcoreMesh(core_axis_name="c", subcore_axis_name="s")
    k = pl.kernel(
        _sck1_body,
        out_type=[
            jax.ShapeDtypeStruct((NW, CH * K), jnp.float32),
            jax.ShapeDtypeStruct((NW, NP), jnp.float32),
        ],
        mesh=mesh,
        scratch_types=[
            pltpu.VMEM((NP,), jnp.float32),
            pltpu.VMEM((NP,), jnp.float32),
            pltpu.VMEM((16,), jnp.float32),
            pltpu.VMEM((CH * K,), jnp.int32),
            pltpu.VMEM((CH * K,), jnp.int32),
            pltpu.VMEM((CH * K,), jnp.float32),
            pltpu.VMEM((NP,), jnp.float32),
        ],
        compiler_params=pltpu.CompilerParams(needs_layout_passes=False),
    )
    return k(asrc, adst, ms16, srcf, dstf)


def _sck2_body(h_hbm, src_hbm, dst_hbm, ex_hbm, zer_hbm, acc_out,
               srcs, dsts, exsl, rows0, acc_s, gs0):
    cid = lax.axis_index("c")
    sid = lax.axis_index("s")
    wid = cid * 16 + sid

    pltpu.sync_copy(zer_hbm, acc_s.at[pl.ds(sid * ROWS_PER_TILE, ROWS_PER_TILE)])
    plsc.subcore_barrier()

    def slab_body(s, _):
        pltpu.sync_copy(src_hbm.at[wid, s], srcs)
        pltpu.sync_copy(dst_hbm.at[wid, s], dsts)
        pltpu.sync_copy(ex_hbm.at[wid, s], exsl)

        def chunk_body(r, _):
            pltpu.async_copy(h_hbm.at[srcs.at[r]], rows0, gs0).wait()
            rbase = jnp.zeros((16,), jnp.int32) + r * K

            def scale_row(j, _):
                exj = plsc.load_gather(exsl, [rbase + j])
                for c in range(D // 16):
                    sl = pl.ds(c * 16, 16)
                    rows0[j, sl] = rows0[j, sl] * exj
                return _

            lax.fori_loop(0, K, scale_row, None)
            pltpu.sync_copy(rows0, acc_s.at[dsts.at[r]], add=True)
            return _

        lax.fori_loop(0, SLAB, chunk_body, None)
        return _

    lax.fori_loop(0, CH // SLAB, slab_body, None)
    plsc.subcore_barrier()
    pltpu.sync_copy(acc_s.at[pl.ds(sid * ROWS_PER_TILE, ROWS_PER_TILE)],
                    acc_out.at[cid, pl.ds(sid * ROWS_PER_TILE, ROWS_PER_TILE)])


def _sc_vector_pass(h, srcr, dstr, ex, zer):
    mesh = plsc.VectorSubcoreMesh(core_axis_name="c", subcore_axis_name="s")
    k = pl.kernel(
        _sck2_body,
        out_type=jax.ShapeDtypeStruct((2, NP, D), jnp.float32),
        mesh=mesh,
        scratch_types=[
            pltpu.VMEM((SLAB, K), jnp.int32),
            pltpu.VMEM((SLAB, K), jnp.int32),
            pltpu.VMEM((SLAB * K,), jnp.float32),
            pltpu.VMEM((K, D), jnp.float32),
            pltpu.VMEM_SHARED((NP, D), jnp.float32),
            pltpu.SemaphoreType.DMA,
        ],
        compiler_params=pltpu.CompilerParams(needs_layout_passes=False),
    )
    return k(h, srcr.reshape(NW, CH // SLAB, SLAB, K),
             dstr.reshape(NW, CH // SLAB, SLAB, K),
             ex.reshape(NW, CH // SLAB, SLAB * K), zer)


def _t3_body(acc_ref, den_ref, b_ref, out_ref):
    i = pl.program_id(0)
    a = acc_ref[0] + acc_ref[1]
    den = jnp.sum(den_ref[...], axis=0)[:, None]
    y = a / (den + 1e-16) + b_ref[...]
    y = jnp.maximum(y, 0.0)
    rows = i * RB + lax.broadcasted_iota(jnp.int32, (RB, 1), 0)
    out_ref[...] = jnp.where(rows < N, y, 0.0)


def _t3(acc, den, b):
    return pl.pallas_call(
        _t3_body,
        grid=(NP // RB,),
        in_specs=[
            pl.BlockSpec((2, RB, D), lambda i: (0, i, 0)),
            pl.BlockSpec((NW, RB), lambda i: (0, i)),
            pl.BlockSpec((1, D), lambda i: (0, 0)),
        ],
        out_specs=pl.BlockSpec((RB, D), lambda i: (i, 0)),
        out_shape=jax.ShapeDtypeStruct((NP, D), jnp.float32),
    )(acc, den, b)


def _layer(xp, srcr, dstr, srcf, dstf, zer, w, a_s, a_d, b):
    h, asrc, adst, ms = _t1(xp, w, a_s.reshape(1, D), a_d.reshape(1, D))
    ms16 = jnp.broadcast_to(ms.reshape(()), (16,))
    ex, den = _sc_scalar_pass(asrc.reshape(NP), adst.reshape(NP), ms16,
                              srcf, dstf)
    acc = _sc_vector_pass(h, srcr, dstr, ex.reshape(NW, CH, K), zer)
    return _t3(acc, den, b.reshape(1, D))


def kernel(x, edge_index, W1, a_src1, a_dst1, b1, W2, a_src2, a_dst2, b2):
    x = x.astype(jnp.float32)
    xp = jnp.zeros((NP, D), jnp.float32).at[:N].set(x)
    loop = jnp.arange(N, dtype=edge_index.dtype)
    ei = jnp.concatenate([edge_index, jnp.stack([loop, loop])], axis=1)
    ei = jnp.pad(ei, ((0, 0), (0, EP - ei.shape[1])), constant_values=N)
    srcr = ei[0].reshape(NW, CH, K)
    dstr = ei[1].reshape(NW, CH, K)
    srcf = ei[0].reshape(NW, CH * K)
    dstf = ei[1].reshape(NW, CH * K)
    zer = jnp.zeros((ROWS_PER_TILE, D), jnp.float32)

    h1 = _layer(xp, srcr, dstr, srcf, dstf, zer, W1, a_src1, a_dst1, b1)
    h2 = _layer(h1, srcr, dstr, srcf, dstf, zer, W2, a_src2, a_dst2, b2)
    return h2[:N]


# speedup vs baseline: 2.6220x; 2.6220x over previous
import functools

import jax
import jax.numpy as jnp
from jax import lax
from jax.experimental import pallas as pl
from jax.experimental.pallas import tpu as pltpu
from jax.experimental.pallas import tpu_sc as plsc

N = 10000
D = 128
NP = 10240
RB = 1280
ROWS_PER_TILE = NP // 16

K = 128
NW = 32
CH = 84
SLAB = 28
EP = NW * CH * K


def _t1_body(x_ref, w_ref, as_ref, ad_ref, h_ref, asrc_ref, adst_ref, ms_ref):
    i = pl.program_id(0)
    h = jnp.dot(x_ref[...], w_ref[...], preferred_element_type=jnp.float32)
    h_ref[...] = h
    asv = jnp.sum(h * as_ref[...], axis=1, keepdims=True)
    adv = jnp.sum(h * ad_ref[...], axis=1, keepdims=True)
    asrc_ref[...] = asv
    adst_ref[...] = adv

    @pl.when(i == 0)
    def _():
        ms_ref[...] = jnp.full((1, 1), -jnp.inf, jnp.float32)

    ms_ref[...] = jnp.maximum(ms_ref[...], jnp.max(asv))


def _t1(x, w, a_s, a_d):
    return pl.pallas_call(
        _t1_body,
        grid=(NP // RB,),
        in_specs=[
            pl.BlockSpec((RB, D), lambda i: (i, 0)),
            pl.BlockSpec((D, D), lambda i: (0, 0)),
            pl.BlockSpec((1, D), lambda i: (0, 0)),
            pl.BlockSpec((1, D), lambda i: (0, 0)),
        ],
        out_specs=[
            pl.BlockSpec((RB, D), lambda i: (i, 0)),
            pl.BlockSpec((RB, 1), lambda i: (i, 0)),
            pl.BlockSpec((RB, 1), lambda i: (i, 0)),
            pl.BlockSpec((1, 1), lambda i: (0, 0)),
        ],
        out_shape=[
            jax.ShapeDtypeStruct((NP, D), jnp.float32),
            jax.ShapeDtypeStruct((NP, 1), jnp.float32),
            jax.ShapeDtypeStruct((NP, 1), jnp.float32),
            jax.ShapeDtypeStruct((1, 1), jnp.float32),
        ],
    )(x, w, a_s, a_d)


def _lrelu(v):
    return jnp.where(v > 0, v, 0.2 * v)


def _sck1_body(asrc_hbm, adst_hbm, ms_hbm, src_hbm, dst_hbm, ex_out,
               den_out, asrc_v, adst_v, ms_v, srcb, dstb, exb, den_v):
    cid = lax.axis_index("c")
    sid = lax.axis_index("s")
    wid = cid * 16 + sid

    pltpu.sync_copy(asrc_hbm, asrc_v)
    pltpu.sync_copy(adst_hbm, adst_v)
    pltpu.sync_copy(ms_hbm, ms_v)
    pltpu.sync_copy(src_hbm.at[wid], srcb)
    pltpu.sync_copy(dst_hbm.at[wid], dstb)

    def zden(i, _):
        den_v[pl.ds(i * 16, 16)] = jnp.zeros((16,), jnp.float32)
        return _

    lax.fori_loop(0, NP // 16, zden, None)

    ms_vec = ms_v[...]

    def grp_body(g, _):
        s16 = srcb[pl.ds(g * 16, 16)]
        d16 = dstb[pl.ds(g * 16, 16)]
        a_s = plsc.load_gather(asrc_v, [s16])
        a_d = plsc.load_gather(adst_v, [d16])
        e = _lrelu(a_s + a_d)
        shift = _lrelu(ms_vec + a_d)
        ex = jnp.exp(e - shift)
        exb[pl.ds(g * 16, 16)] = ex
        plsc.addupdate_scatter(den_v, [d16], ex)
        return _

    lax.fori_loop(0, (CH * K) // 16, grp_body, None)
    pltpu.sync_copy(exb, ex_out.at[wid])
    pltpu.sync_copy(den_v, den_out.at[wid])


def _sc_scalar_pass(asrc, adst, ms16, srcf, dstf):
    mesh = plsc.VectorSubcoreMesh(core_axis_name="c", subcore_axis_name="s")
    k = pl.kernel(
        _sck1_body,
        out_type=[
            jax.ShapeDtypeStruct((NW, CH * K), jnp.float32),
            jax.ShapeDtypeStruct((NW, NP), jnp.float32),
        ],
        mesh=mesh,
        scratch_types=[
            pltpu.VMEM((NP,), jnp.float32),
            pltpu.VMEM((NP,), jnp.float32),
            pltpu.VMEM((16,), jnp.float32),
            pltpu.VMEM((CH * K,), jnp.int32),
            pltpu.VMEM((CH * K,), jnp.int32),
            pltpu.VMEM((CH * K,), jnp.float32),
            pltpu.VMEM((NP,), jnp.float32),
        ],
        compiler_params=pltpu.CompilerParams(needs_layout_passes=False),
    )
    return k(asrc, adst, ms16, srcf, dstf)


def _sck2_body(h_hbm, src_hbm, dst_hbm, ex_hbm, zer_hbm, acc_out,
               srcs, dsts, exsl, rows0, acc_s, gs0):
    cid = lax.axis_index("c")
    sid = lax.axis_index("s")
    wid = cid * 16 + sid

    pltpu.sync_copy(zer_hbm, acc_s.at[pl.ds(sid * ROWS_PER_TILE, ROWS_PER_TILE)])
    plsc.subcore_barrier()

    def slab_body(s, _):
        pltpu.sync_copy(src_hbm.at[wid, s], srcs)
        pltpu.sync_copy(dst_hbm.at[wid, s], dsts)
        pltpu.sync_copy(ex_hbm.at[wid, s], exsl)

        def chunk_body(r, _):
            pltpu.async_copy(h_hbm.at[srcs.at[r]], rows0, gs0).wait()
            rbase = jnp.zeros((16,), jnp.int32) + r * K

            def scale_row(j, _):
                exj = plsc.load_gather(exsl, [rbase + j])
                for c in range(D // 16):
                    sl = pl.ds(c * 16, 16)
                    rows0[j, sl] = rows0[j, sl] * exj
                return _

            lax.fori_loop(0, K, scale_row, None)
            pltpu.sync_copy(rows0, acc_s.at[dsts.at[r]], add=True)
            return _

        lax.fori_loop(0, SLAB, chunk_body, None)
        return _

    lax.fori_loop(0, CH // SLAB, slab_body, None)
    plsc.subcore_barrier()
    pltpu.sync_copy(acc_s.at[pl.ds(sid * ROWS_PER_TILE, ROWS_PER_TILE)],
                    acc_out.at[cid, pl.ds(sid * ROWS_PER_TILE, ROWS_PER_TILE)])


def _sc_vector_pass(h, srcr, dstr, ex, zer):
    mesh = plsc.VectorSubcoreMesh(core_axis_name="c", subcore_axis_name="s")
    k = pl.kernel(
        _sck2_body,
        out_type=jax.ShapeDtypeStruct((2, NP, D), jnp.float32),
        mesh=mesh,
        scratch_types=[
            pltpu.VMEM((SLAB, K), jnp.int32),
            pltpu.VMEM((SLAB, K), jnp.int32),
            pltpu.VMEM((SLAB * K,), jnp.float32),
            pltpu.VMEM((K, D), jnp.float32),
            pltpu.VMEM_SHARED((NP, D), jnp.float32),
            pltpu.SemaphoreType.DMA,
        ],
        compiler_params=pltpu.CompilerParams(needs_layout_passes=False),
    )
    return k(h, srcr.reshape(NW, CH // SLAB, SLAB, K),
             dstr.reshape(NW, CH // SLAB, SLAB, K),
             ex.reshape(NW, CH // SLAB, SLAB * K), zer)


def _t3_body(acc_ref, den_ref, b_ref, out_ref):
    i = pl.program_id(0)
    a = acc_ref[0] + acc_ref[1]
    den = jnp.sum(den_ref[...], axis=0)[:, None]
    y = a / (den + 1e-16) + b_ref[...]
    y = jnp.maximum(y, 0.0)
    rows = i * RB + lax.broadcasted_iota(jnp.int32, (RB, 1), 0)
    out_ref[...] = jnp.where(rows < N, y, 0.0)


def _t3(acc, den, b):
    return pl.pallas_call(
        _t3_body,
        grid=(NP // RB,),
        in_specs=[
            pl.BlockSpec((2, RB, D), lambda i: (0, i, 0)),
            pl.BlockSpec((NW, RB), lambda i: (0, i)),
            pl.BlockSpec((1, D), lambda i: (0, 0)),
        ],
        out_specs=pl.BlockSpec((RB, D), lambda i: (i, 0)),
        out_shape=jax.ShapeDtypeStruct((NP, D), jnp.float32),
    )(acc, den, b)


def _layer(xp, srcr, dstr, srcf, dstf, zer, w, a_s, a_d, b):
    h, asrc, adst, ms = _t1(xp, w, a_s.reshape(1, D), a_d.reshape(1, D))
    ms16 = jnp.broadcast_to(ms.reshape(()), (16,))
    ex, den = _sc_scalar_pass(asrc.reshape(NP), adst.reshape(NP), ms16,
                              srcf, dstf)
    acc = _sc_vector_pass(h, srcr, dstr, ex.reshape(NW, CH, K), zer)
    return _t3(acc, den, b.reshape(1, D))


def kernel(x, edge_index, W1, a_src1, a_dst1, b1, W2, a_src2, a_dst2, b2):
    x = x.astype(jnp.float32)
    xp = jnp.zeros((NP, D), jnp.float32).at[:N].set(x)
    loop = jnp.arange(N, dtype=edge_index.dtype)
    ei = jnp.concatenate([edge_index, jnp.stack([loop, loop])], axis=1)
    npad = EP - ei.shape[1]
    padv = N + (jnp.arange(npad, dtype=ei.dtype) % (NP - N))
    ei = jnp.concatenate([ei, jnp.stack([padv, padv])], axis=1)
    srcr = ei[0].reshape(NW, CH, K)
    dstr = ei[1].reshape(NW, CH, K)
    srcf = ei[0].reshape(NW, CH * K)
    dstf = ei[1].reshape(NW, CH * K)
    zer = jnp.zeros((ROWS_PER_TILE, D), jnp.float32)

    h1 = _layer(xp, srcr, dstr, srcf, dstf, zer, W1, a_src1, a_dst1, b1)
    h2 = _layer(h1, srcr, dstr, srcf, dstf, zer, W2, a_src2, a_dst2, b2)
    return h2[:N]


# speedup vs baseline: 3.6106x; 1.3771x over previous
import functools

import jax
import jax.numpy as jnp
from jax import lax
from jax.experimental import pallas as pl
from jax.experimental.pallas import tpu as pltpu
from jax.experimental.pallas import tpu_sc as plsc

N = 10000
D = 128
NP = 10240
RB = 1280
ROWS_PER_TILE = NP // 16

K = 128
NW = 32
CH = 84
SLAB = 28
EP = NW * CH * K


def _t1_body(x_ref, w_ref, as_ref, ad_ref, h_ref, asrc_ref, adst_ref, ms_ref):
    i = pl.program_id(0)
    h = jnp.dot(x_ref[...], w_ref[...], preferred_element_type=jnp.float32)
    h_ref[...] = h
    asv = jnp.sum(h * as_ref[...], axis=1, keepdims=True)
    adv = jnp.sum(h * ad_ref[...], axis=1, keepdims=True)
    asrc_ref[...] = asv
    adst_ref[...] = adv

    @pl.when(i == 0)
    def _():
        ms_ref[...] = jnp.full((1, 1), -jnp.inf, jnp.float32)

    ms_ref[...] = jnp.maximum(ms_ref[...], jnp.max(asv))


def _t1(x, w, a_s, a_d):
    return pl.pallas_call(
        _t1_body,
        grid=(NP // RB,),
        in_specs=[
            pl.BlockSpec((RB, D), lambda i: (i, 0)),
            pl.BlockSpec((D, D), lambda i: (0, 0)),
            pl.BlockSpec((1, D), lambda i: (0, 0)),
            pl.BlockSpec((1, D), lambda i: (0, 0)),
        ],
        out_specs=[
            pl.BlockSpec((RB, D), lambda i: (i, 0)),
            pl.BlockSpec((RB, 1), lambda i: (i, 0)),
            pl.BlockSpec((RB, 1), lambda i: (i, 0)),
            pl.BlockSpec((1, 1), lambda i: (0, 0)),
        ],
        out_shape=[
            jax.ShapeDtypeStruct((NP, D), jnp.float32),
            jax.ShapeDtypeStruct((NP, 1), jnp.float32),
            jax.ShapeDtypeStruct((NP, 1), jnp.float32),
            jax.ShapeDtypeStruct((1, 1), jnp.float32),
        ],
    )(x, w, a_s, a_d)


def _lrelu(v):
    return jnp.where(v > 0, v, 0.2 * v)


def _sck1_body(asrc_hbm, adst_hbm, ms_hbm, src_hbm, dst_hbm, ex_out,
               den_out, asrc_v, adst_v, ms_v, srcb, dstb, exb, den_v):
    cid = lax.axis_index("c")
    sid = lax.axis_index("s")
    wid = cid * 16 + sid

    pltpu.sync_copy(asrc_hbm, asrc_v)
    pltpu.sync_copy(adst_hbm, adst_v)
    pltpu.sync_copy(ms_hbm, ms_v)
    pltpu.sync_copy(src_hbm.at[wid], srcb)
    pltpu.sync_copy(dst_hbm.at[wid], dstb)

    def zden(i, _):
        den_v[pl.ds(i * 16, 16)] = jnp.zeros((16,), jnp.float32)
        return _

    lax.fori_loop(0, NP // 16, zden, None)

    ms_vec = ms_v[...]

    def grp_body(g, _):
        s16 = srcb[pl.ds(g * 16, 16)]
        d16 = dstb[pl.ds(g * 16, 16)]
        a_s = plsc.load_gather(asrc_v, [s16])
        a_d = plsc.load_gather(adst_v, [d16])
        e = _lrelu(a_s + a_d)
        shift = _lrelu(ms_vec + a_d)
        ex = jnp.exp(e - shift)
        exb[pl.ds(g * 16, 16)] = ex
        plsc.addupdate_scatter(den_v, [d16], ex)
        return _

    lax.fori_loop(0, (CH * K) // 16, grp_body, None)
    pltpu.sync_copy(exb, ex_out.at[wid])
    pltpu.sync_copy(den_v, den_out.at[wid])


def _sc_scalar_pass(asrc, adst, ms16, srcf, dstf):
    mesh = plsc.VectorSubcoreMesh(core_axis_name="c", subcore_axis_name="s")
    k = pl.kernel(
        _sck1_body,
        out_type=[
            jax.ShapeDtypeStruct((NW, CH * K), jnp.float32),
            jax.ShapeDtypeStruct((NW, NP), jnp.float32),
        ],
        mesh=mesh,
        scratch_types=[
            pltpu.VMEM((NP,), jnp.float32),
            pltpu.VMEM((NP,), jnp.float32),
            pltpu.VMEM((16,), jnp.float32),
            pltpu.VMEM((CH * K,), jnp.int32),
            pltpu.VMEM((CH * K,), jnp.int32),
            pltpu.VMEM((CH * K,), jnp.float32),
            pltpu.VMEM((NP,), jnp.float32),
        ],
        compiler_params=pltpu.CompilerParams(needs_layout_passes=False),
    )
    return k(asrc, adst, ms16, srcf, dstf)


def _sck2_body(h_hbm, src_hbm, dst_hbm, ex_hbm, zer_hbm, acc_out,
               srcs, dsts, exsl, rows0, rows1, acc_s, gs0, gs1):
    cid = lax.axis_index("c")
    sid = lax.axis_index("s")
    wid = cid * 16 + sid

    pltpu.sync_copy(zer_hbm, acc_s.at[pl.ds(sid * ROWS_PER_TILE, ROWS_PER_TILE)])
    plsc.subcore_barrier()

    rows = (rows0, rows1)
    gsem = (gs0, gs1)

    def gath(r, b):
        return pltpu.make_async_copy(h_hbm.at[srcs.at[r]], rows[b], gsem[b])

    def slab_body(s, _):
        pltpu.sync_copy(src_hbm.at[wid, s], srcs)
        pltpu.sync_copy(dst_hbm.at[wid, s], dsts)
        pltpu.sync_copy(ex_hbm.at[wid, s], exsl)
        gath(0, 0).start()

        def pair_body(p, _):
            for b in range(2):
                r = 2 * p + b

                @pl.when(r < SLAB - 1)
                def _():
                    gath(r + 1, 1 - b).start()

                gath(r, b).wait()
                rbase = jnp.zeros((16,), jnp.int32) + r * K

                def scale_row(j, _):
                    exj = plsc.load_gather(exsl, [rbase + j])
                    for c in range(D // 16):
                        sl = pl.ds(c * 16, 16)
                        rows[b][j, sl] = rows[b][j, sl] * exj
                    return _

                lax.fori_loop(0, K, scale_row, None)
                pltpu.sync_copy(rows[b], acc_s.at[dsts.at[r]], add=True)
            return _

        lax.fori_loop(0, SLAB // 2, pair_body, None)
        return _

    lax.fori_loop(0, CH // SLAB, slab_body, None)
    plsc.subcore_barrier()
    pltpu.sync_copy(acc_s.at[pl.ds(sid * ROWS_PER_TILE, ROWS_PER_TILE)],
                    acc_out.at[cid, pl.ds(sid * ROWS_PER_TILE, ROWS_PER_TILE)])


def _sc_vector_pass(h, srcr, dstr, ex, zer):
    mesh = plsc.VectorSubcoreMesh(core_axis_name="c", subcore_axis_name="s")
    k = pl.kernel(
        _sck2_body,
        out_type=jax.ShapeDtypeStruct((2, NP, D), jnp.float32),
        mesh=mesh,
        scratch_types=[
            pltpu.VMEM((SLAB, K), jnp.int32),
            pltpu.VMEM((SLAB, K), jnp.int32),
            pltpu.VMEM((SLAB * K,), jnp.float32),
            pltpu.VMEM((K, D), jnp.float32),
            pltpu.VMEM((K, D), jnp.float32),
            pltpu.VMEM_SHARED((NP, D), jnp.float32),
            pltpu.SemaphoreType.DMA,
            pltpu.SemaphoreType.DMA,
        ],
        compiler_params=pltpu.CompilerParams(needs_layout_passes=False),
    )
    return k(h, srcr.reshape(NW, CH // SLAB, SLAB, K),
             dstr.reshape(NW, CH // SLAB, SLAB, K),
             ex.reshape(NW, CH // SLAB, SLAB * K), zer)


def _t3_body(acc_ref, den_ref, b_ref, out_ref):
    i = pl.program_id(0)
    a = acc_ref[0] + acc_ref[1]
    den = jnp.sum(den_ref[...], axis=0)[:, None]
    y = a / (den + 1e-16) + b_ref[...]
    y = jnp.maximum(y, 0.0)
    rows = i * RB + lax.broadcasted_iota(jnp.int32, (RB, 1), 0)
    out_ref[...] = jnp.where(rows < N, y, 0.0)


def _t3(acc, den, b):
    return pl.pallas_call(
        _t3_body,
        grid=(NP // RB,),
        in_specs=[
            pl.BlockSpec((2, RB, D), lambda i: (0, i, 0)),
            pl.BlockSpec((NW, RB), lambda i: (0, i)),
            pl.BlockSpec((1, D), lambda i: (0, 0)),
        ],
        out_specs=pl.BlockSpec((RB, D), lambda i: (i, 0)),
        out_shape=jax.ShapeDtypeStruct((NP, D), jnp.float32),
    )(acc, den, b)


def _layer(xp, srcr, dstr, srcf, dstf, zer, w, a_s, a_d, b):
    h, asrc, adst, ms = _t1(xp, w, a_s.reshape(1, D), a_d.reshape(1, D))
    ms16 = jnp.broadcast_to(ms.reshape(()), (16,))
    ex, den = _sc_scalar_pass(asrc.reshape(NP), adst.reshape(NP), ms16,
                              srcf, dstf)
    acc = _sc_vector_pass(h, srcr, dstr, ex.reshape(NW, CH, K), zer)
    return _t3(acc, den, b.reshape(1, D))


def kernel(x, edge_index, W1, a_src1, a_dst1, b1, W2, a_src2, a_dst2, b2):
    x = x.astype(jnp.float32)
    xp = jnp.zeros((NP, D), jnp.float32).at[:N].set(x)
    loop = jnp.arange(N, dtype=edge_index.dtype)
    ei = jnp.concatenate([edge_index, jnp.stack([loop, loop])], axis=1)
    npad = EP - ei.shape[1]
    padv = N + (jnp.arange(npad, dtype=ei.dtype) % (NP - N))
    ei = jnp.concatenate([ei, jnp.stack([padv, padv])], axis=1)
    srcr = ei[0].reshape(NW, CH, K)
    dstr = ei[1].reshape(NW, CH, K)
    srcf = ei[0].reshape(NW, CH * K)
    dstf = ei[1].reshape(NW, CH * K)
    zer = jnp.zeros((ROWS_PER_TILE, D), jnp.float32)

    h1 = _layer(xp, srcr, dstr, srcf, dstf, zer, W1, a_src1, a_dst1, b1)
    h2 = _layer(h1, srcr, dstr, srcf, dstf, zer, W2, a_src2, a_dst2, b2)
    return h2[:N]


# speedup vs baseline: 4.2286x; 1.1712x over previous
import functools

import jax
import jax.numpy as jnp
from jax import lax
from jax.experimental import pallas as pl
from jax.experimental.pallas import tpu as pltpu
from jax.experimental.pallas import tpu_sc as plsc

N = 10000
D = 128
NP = 10240
RB = 1280
ROWS_PER_TILE = NP // 16

K = 96
NW = 32
SLAB = 27
NSLAB = 4
CH = SLAB * NSLAB
EP = NW * CH * K


def _t1_body(x_ref, w_ref, as_ref, ad_ref, h_ref, asrc_ref, adst_ref, ms_ref):
    i = pl.program_id(0)
    h = jnp.dot(x_ref[...], w_ref[...], preferred_element_type=jnp.float32)
    h_ref[...] = h
    asv = jnp.sum(h * as_ref[...], axis=1, keepdims=True)
    adv = jnp.sum(h * ad_ref[...], axis=1, keepdims=True)
    asrc_ref[...] = asv
    adst_ref[...] = adv

    @pl.when(i == 0)
    def _():
        ms_ref[...] = jnp.full((1, 1), -jnp.inf, jnp.float32)

    ms_ref[...] = jnp.maximum(ms_ref[...], jnp.max(asv))


def _t1(x, w, a_s, a_d):
    return pl.pallas_call(
        _t1_body,
        grid=(NP // RB,),
        in_specs=[
            pl.BlockSpec((RB, D), lambda i: (i, 0)),
            pl.BlockSpec((D, D), lambda i: (0, 0)),
            pl.BlockSpec((1, D), lambda i: (0, 0)),
            pl.BlockSpec((1, D), lambda i: (0, 0)),
        ],
        out_specs=[
            pl.BlockSpec((RB, D), lambda i: (i, 0)),
            pl.BlockSpec((RB, 1), lambda i: (i, 0)),
            pl.BlockSpec((RB, 1), lambda i: (i, 0)),
            pl.BlockSpec((1, 1), lambda i: (0, 0)),
        ],
        out_shape=[
            jax.ShapeDtypeStruct((NP, D), jnp.float32),
            jax.ShapeDtypeStruct((NP, 1), jnp.float32),
            jax.ShapeDtypeStruct((NP, 1), jnp.float32),
            jax.ShapeDtypeStruct((1, 1), jnp.float32),
        ],
    )(x, w, a_s, a_d)


def _lrelu(v):
    return jnp.where(v > 0, v, 0.2 * v)


def _sck1_body(asrc_hbm, adst_hbm, ms_hbm, src_hbm, dst_hbm, ex_out,
               den_out, asrc_v, adst_v, ms_v, srcb, dstb, exb, den_v):
    cid = lax.axis_index("c")
    sid = lax.axis_index("s")
    wid = cid * 16 + sid

    pltpu.sync_copy(asrc_hbm, asrc_v)
    pltpu.sync_copy(adst_hbm, adst_v)
    pltpu.sync_copy(ms_hbm, ms_v)
    pltpu.sync_copy(src_hbm.at[wid], srcb)
    pltpu.sync_copy(dst_hbm.at[wid], dstb)

    def zden(i, _):
        den_v[pl.ds(i * 16, 16)] = jnp.zeros((16,), jnp.float32)
        return _

    lax.fori_loop(0, NP // 16, zden, None)

    ms_vec = ms_v[...]

    def grp_body(g, _):
        s16 = srcb[pl.ds(g * 16, 16)]
        d16 = dstb[pl.ds(g * 16, 16)]
        a_s = plsc.load_gather(asrc_v, [s16])
        a_d = plsc.load_gather(adst_v, [d16])
        e = _lrelu(a_s + a_d)
        shift = _lrelu(ms_vec + a_d)
        ex = jnp.exp(e - shift)
        exb[pl.ds(g * 16, 16)] = ex
        plsc.addupdate_scatter(den_v, [d16], ex)
        return _

    lax.fori_loop(0, (CH * K) // 16, grp_body, None)
    pltpu.sync_copy(exb, ex_out.at[wid])
    pltpu.sync_copy(den_v, den_out.at[wid])


def _sc_scalar_pass(asrc, adst, ms16, srcf, dstf):
    mesh = plsc.VectorSubcoreMesh(core_axis_name="c", subcore_axis_name="s")
    k = pl.kernel(
        _sck1_body,
        out_type=[
            jax.ShapeDtypeStruct((NW, CH * K), jnp.float32),
            jax.ShapeDtypeStruct((NW, NP), jnp.float32),
        ],
        mesh=mesh,
        scratch_types=[
            pltpu.VMEM((NP,), jnp.float32),
            pltpu.VMEM((NP,), jnp.float32),
            pltpu.VMEM((16,), jnp.float32),
            pltpu.VMEM((CH * K,), jnp.int32),
            pltpu.VMEM((CH * K,), jnp.int32),
            pltpu.VMEM((CH * K,), jnp.float32),
            pltpu.VMEM((NP,), jnp.float32),
        ],
        compiler_params=pltpu.CompilerParams(needs_layout_passes=False),
    )
    return k(asrc, adst, ms16, srcf, dstf)


def _sck2_body(h_hbm, src_hbm, dst_hbm, ex_hbm, zer_hbm, acc_out,
               srcs, dsts, exsl, rows0, rows1, rows2, acc_s,
               gs0, gs1, gs2, ss0, ss1, ss2):
    cid = lax.axis_index("c")
    sid = lax.axis_index("s")
    wid = cid * 16 + sid

    pltpu.sync_copy(zer_hbm, acc_s.at[pl.ds(sid * ROWS_PER_TILE, ROWS_PER_TILE)])
    plsc.subcore_barrier()

    rows = (rows0, rows1, rows2)
    gsem = (gs0, gs1, gs2)
    ssem = (ss0, ss1, ss2)

    def gath(r, b):
        return pltpu.make_async_copy(h_hbm.at[srcs.at[r]], rows[b], gsem[b])

    def scat(r, b):
        return pltpu.make_async_copy(rows[b], acc_s.at[dsts.at[r]], ssem[b])

    def slab_body(s, _):
        pltpu.sync_copy(src_hbm.at[wid, s], srcs)
        pltpu.sync_copy(dst_hbm.at[wid, s], dsts)
        pltpu.sync_copy(ex_hbm.at[wid, s], exsl)
        gath(0, 0).start()

        def grp_body(p, _):
            for b in range(3):
                r = 3 * p + b

                @pl.when(r >= 2)
                def _():
                    scat(r - 2, (b + 1) % 3).wait()

                @pl.when(r < SLAB - 1)
                def _():
                    gath(r + 1, (b + 1) % 3).start()

                gath(r, b).wait()
                rbase = jnp.zeros((16,), jnp.int32) + r * K

                def scale_row(jj, _):
                    for u in range(2):
                        j = 2 * jj + u
                        exj = plsc.load_gather(exsl, [rbase + j])
                        for c in range(D // 16):
                            sl = pl.ds(c * 16, 16)
                            rows[b][j, sl] = rows[b][j, sl] * exj
                    return _

                lax.fori_loop(0, K // 2, scale_row, None)
                scat(r, b).start(add=True)
            return _

        lax.fori_loop(0, SLAB // 3, grp_body, None)
        scat(SLAB - 2, (SLAB - 2) % 3).wait()
        scat(SLAB - 1, (SLAB - 1) % 3).wait()
        return _

    lax.fori_loop(0, NSLAB, slab_body, None)
    plsc.subcore_barrier()
    pltpu.sync_copy(acc_s.at[pl.ds(sid * ROWS_PER_TILE, ROWS_PER_TILE)],
                    acc_out.at[cid, pl.ds(sid * ROWS_PER_TILE, ROWS_PER_TILE)])


def _sc_vector_pass(h, srcr, dstr, ex, zer):
    mesh = plsc.VectorSubcoreMesh(core_axis_name="c", subcore_axis_name="s")
    k = pl.kernel(
        _sck2_body,
        out_type=jax.ShapeDtypeStruct((2, NP, D), jnp.float32),
        mesh=mesh,
        scratch_types=[
            pltpu.VMEM((SLAB, K), jnp.int32),
            pltpu.VMEM((SLAB, K), jnp.int32),
            pltpu.VMEM((SLAB * K,), jnp.float32),
            pltpu.VMEM((K, D), jnp.float32),
            pltpu.VMEM((K, D), jnp.float32),
            pltpu.VMEM((K, D), jnp.float32),
            pltpu.VMEM_SHARED((NP, D), jnp.float32),
            pltpu.SemaphoreType.DMA,
            pltpu.SemaphoreType.DMA,
            pltpu.SemaphoreType.DMA,
            pltpu.SemaphoreType.DMA,
            pltpu.SemaphoreType.DMA,
            pltpu.SemaphoreType.DMA,
        ],
        compiler_params=pltpu.CompilerParams(needs_layout_passes=False),
    )
    return k(h, srcr.reshape(NW, NSLAB, SLAB, K),
             dstr.reshape(NW, NSLAB, SLAB, K),
             ex.reshape(NW, NSLAB, SLAB * K), zer)


def _t3_body(acc_ref, den_ref, b_ref, out_ref):
    i = pl.program_id(0)
    a = acc_ref[0] + acc_ref[1]
    den = jnp.sum(den_ref[...], axis=0)[:, None]
    y = a / (den + 1e-16) + b_ref[...]
    y = jnp.maximum(y, 0.0)
    rows = i * RB + lax.broadcasted_iota(jnp.int32, (RB, 1), 0)
    out_ref[...] = jnp.where(rows < N, y, 0.0)


def _t3(acc, den, b):
    return pl.pallas_call(
        _t3_body,
        grid=(NP // RB,),
        in_specs=[
            pl.BlockSpec((2, RB, D), lambda i: (0, i, 0)),
            pl.BlockSpec((NW, RB), lambda i: (0, i)),
            pl.BlockSpec((1, D), lambda i: (0, 0)),
        ],
        out_specs=pl.BlockSpec((RB, D), lambda i: (i, 0)),
        out_shape=jax.ShapeDtypeStruct((NP, D), jnp.float32),
    )(acc, den, b)


def _layer(xp, srcr, dstr, srcf, dstf, zer, w, a_s, a_d, b):
    h, asrc, adst, ms = _t1(xp, w, a_s.reshape(1, D), a_d.reshape(1, D))
    ms16 = jnp.broadcast_to(ms.reshape(()), (16,))
    ex, den = _sc_scalar_pass(asrc.reshape(NP), adst.reshape(NP), ms16,
                              srcf, dstf)
    acc = _sc_vector_pass(h, srcr, dstr, ex.reshape(NW, CH, K), zer)
    return _t3(acc, den, b.reshape(1, D))


def kernel(x, edge_index, W1, a_src1, a_dst1, b1, W2, a_src2, a_dst2, b2):
    x = x.astype(jnp.float32)
    xp = jnp.zeros((NP, D), jnp.float32).at[:N].set(x)
    loop = jnp.arange(N, dtype=edge_index.dtype)
    ei = jnp.concatenate([edge_index, jnp.stack([loop, loop])], axis=1)
    npad = EP - ei.shape[1]
    padv = N + (jnp.arange(npad, dtype=ei.dtype) % (NP - N))
    ei = jnp.concatenate([ei, jnp.stack([padv, padv])], axis=1)
    srcr = ei[0].reshape(NW, CH, K)
    dstr = ei[1].reshape(NW, CH, K)
    srcf = ei[0].reshape(NW, CH * K)
    dstf = ei[1].reshape(NW, CH * K)
    zer = jnp.zeros((ROWS_PER_TILE, D), jnp.float32)

    h1 = _layer(xp, srcr, dstr, srcf, dstf, zer, W1, a_src1, a_dst1, b1)
    h2 = _layer(h1, srcr, dstr, srcf, dstf, zer, W2, a_src2, a_dst2, b2)
    return h2[:N]
